# Initial kernel scaffold; baseline (speedup 1.0000x reference)
import jax
import jax.numpy as jnp
from jax.experimental import pallas as pl


def kernel(X, vertex, edges, degE, degV, W):
    raise NotImplementedError("write your pallas kernel here")


# speedup vs baseline: 6.1474x; 6.1474x over previous
import functools

import jax
import jax.numpy as jnp
from jax import lax
from jax.experimental import pallas as pl
from jax.experimental.pallas import tpu as pltpu
from jax.experimental.pallas import tpu_sc as plsc

NC = 2
NS = 16
NW = NC * NS
CHUNK = 80


def _tc_linear_aug(X, Wt, DA):
    N, D = X.shape
    OUT = Wt.shape[1]
    B = 1000

    def body(x_ref, w_ref, o_ref):
        acc = jnp.dot(x_ref[...], w_ref[...], preferred_element_type=jnp.float32)
        o_ref[:, :OUT] = acc
        j = lax.broadcasted_iota(jnp.int32, (B, DA - OUT), 1)
        o_ref[:, OUT:] = jnp.where(j == 0, jnp.float32(1.0), jnp.float32(0.0))

    return pl.pallas_call(
        body,
        grid=(N // B,),
        in_specs=[pl.BlockSpec((B, D), lambda i: (i, 0)),
                  pl.BlockSpec((D, OUT), lambda i: (0, 0))],
        out_specs=pl.BlockSpec((B, DA), lambda i: (i, 0)),
        out_shape=jax.ShapeDtypeStruct((N, DA), jnp.float32),
    )(X, Wt)


def _tc_mean_scale(partials, degE, OUT):
    _, M, DA = partials.shape
    B = 2000

    def body(p_ref, de_ref, o_ref):
        s = p_ref[0] + p_ref[1]
        cnt = s[:, OUT:OUT + 1]
        o_ref[...] = s[:, :OUT] / jnp.maximum(cnt, 1.0) * de_ref[...]

    return pl.pallas_call(
        body,
        grid=(M // B,),
        in_specs=[pl.BlockSpec((2, B, DA), lambda i: (0, i, 0)),
                  pl.BlockSpec((B, 1), lambda i: (i, 0))],
        out_specs=pl.BlockSpec((B, OUT), lambda i: (i, 0)),
        out_shape=jax.ShapeDtypeStruct((M, OUT), jnp.float32),
    )(partials, degE)


def _tc_scale(partials, degV):
    _, N, OUT = partials.shape
    B = 2000

    def body(p_ref, dv_ref, o_ref):
        o_ref[...] = (p_ref[0] + p_ref[1]) * dv_ref[...]

    return pl.pallas_call(
        body,
        grid=(N // B,),
        in_specs=[pl.BlockSpec((2, B, OUT), lambda i: (0, i, 0)),
                  pl.BlockSpec((B, 1), lambda i: (i, 0))],
        out_specs=pl.BlockSpec((B, OUT), lambda i: (i, 0)),
        out_shape=jax.ShapeDtypeStruct((N, OUT), jnp.float32),
    )(partials, degV)


def _make_sc_stage(R, width, nnz):
    P = nnz // NW
    nchunk = P // CHUNK
    rz = R // NS
    mesh = plsc.VectorSubcoreMesh(core_axis_name="c", subcore_axis_name="s")

    @functools.partial(
        pl.kernel,
        out_type=jax.ShapeDtypeStruct((NC, R, width), jnp.float32),
        mesh=mesh,
        compiler_params=pltpu.CompilerParams(use_tc_tiling_on_sc=False),
        scratch_types=[
            pltpu.VMEM((P,), jnp.int32),
            pltpu.VMEM((CHUNK,), jnp.int32),
            pltpu.VMEM((CHUNK, width), jnp.float32),
            pltpu.VMEM_SHARED((R, width), jnp.float32),
            pltpu.SemaphoreType.DMA,
        ],
    )
    def stage(table, gidx, sidx, zeros, out, gi, si, rows, acc, sem):
        cid = lax.axis_index("c")
        sid = lax.axis_index("s")
        wid = cid * NS + sid
        pltpu.sync_copy(zeros.at[pl.ds(sid * rz, rz)], acc.at[pl.ds(sid * rz, rz)])
        base0 = wid * P
        pltpu.sync_copy(gidx.at[pl.ds(base0, P)], gi)
        plsc.subcore_barrier()

        def body(j, carry):
            pltpu.sync_copy(sidx.at[pl.ds(base0 + j * CHUNK, CHUNK)], si)
            pltpu.async_copy(table.at[gi.at[pl.ds(j * CHUNK, CHUNK)]],
                             rows, sem).wait()
            pltpu.sync_copy(rows, acc.at[si], add=True)
            return carry

        lax.fori_loop(0, nchunk, body, 0)
        plsc.subcore_barrier()
        pltpu.sync_copy(acc.at[pl.ds(sid * rz, rz)],
                        out.at[cid, pl.ds(sid * rz, rz)])

    return stage


def kernel(X, vertex, edges, degE, degV, W):
    N, D = X.shape
    OUT = W.shape[0]
    M = degE.shape[0]
    NNZ = vertex.shape[0]
    DA = OUT + 16

    xp_aug = _tc_linear_aug(X, W.T, DA)

    zA = jnp.zeros((M, DA), jnp.float32)
    pA = _make_sc_stage(M, DA, NNZ)(xp_aug, vertex, edges, zA)
    xe = _tc_mean_scale(pA, degE, OUT)

    zB = jnp.zeros((N, OUT), jnp.float32)
    pB = _make_sc_stage(N, OUT, NNZ)(xe, edges, vertex, zB)
    return _tc_scale(pB, degV)


# speedup vs baseline: 9.3276x; 1.5173x over previous
import functools

import jax
import jax.numpy as jnp
from jax import lax
from jax.experimental import pallas as pl
from jax.experimental.pallas import tpu as pltpu
from jax.experimental.pallas import tpu_sc as plsc

NC = 2
NS = 16
NW = NC * NS
CHUNK = 80


def _tc_linear_aug(X, Wt, DA):
    N, D = X.shape
    OUT = Wt.shape[1]
    B = 1000

    def body(x_ref, w_ref, o_ref):
        acc = jnp.dot(x_ref[...], w_ref[...], preferred_element_type=jnp.float32)
        o_ref[:, :OUT] = acc
        j = lax.broadcasted_iota(jnp.int32, (B, DA - OUT), 1)
        o_ref[:, OUT:] = jnp.where(j == 0, jnp.float32(1.0), jnp.float32(0.0))

    return pl.pallas_call(
        body,
        grid=(N // B,),
        in_specs=[pl.BlockSpec((B, D), lambda i: (i, 0)),
                  pl.BlockSpec((D, OUT), lambda i: (0, 0))],
        out_specs=pl.BlockSpec((B, DA), lambda i: (i, 0)),
        out_shape=jax.ShapeDtypeStruct((N, DA), jnp.float32),
    )(X, Wt)


def _tc_mean_scale(partials, degE, OUT):
    _, M, DA = partials.shape
    B = 2000

    def body(p_ref, de_ref, o_ref):
        s = p_ref[0] + p_ref[1]
        cnt = s[:, OUT:OUT + 1]
        o_ref[...] = s[:, :OUT] / jnp.maximum(cnt, 1.0) * de_ref[...]

    return pl.pallas_call(
        body,
        grid=(M // B,),
        in_specs=[pl.BlockSpec((2, B, DA), lambda i: (0, i, 0)),
                  pl.BlockSpec((B, 1), lambda i: (i, 0))],
        out_specs=pl.BlockSpec((B, OUT), lambda i: (i, 0)),
        out_shape=jax.ShapeDtypeStruct((M, OUT), jnp.float32),
    )(partials, degE)


def _tc_scale(partials, degV):
    _, N, OUT = partials.shape
    B = 2000

    def body(p_ref, dv_ref, o_ref):
        o_ref[...] = (p_ref[0] + p_ref[1]) * dv_ref[...]

    return pl.pallas_call(
        body,
        grid=(N // B,),
        in_specs=[pl.BlockSpec((2, B, OUT), lambda i: (0, i, 0)),
                  pl.BlockSpec((B, 1), lambda i: (i, 0))],
        out_specs=pl.BlockSpec((B, OUT), lambda i: (i, 0)),
        out_shape=jax.ShapeDtypeStruct((N, OUT), jnp.float32),
    )(partials, degV)


def _make_sc_stage(R, width, nnz):
    P = nnz // NW
    nchunk = P // CHUNK
    assert nchunk % 2 == 1, "pipeline below handles an odd chunk count"
    rz = R // NS
    mesh = plsc.VectorSubcoreMesh(core_axis_name="c", subcore_axis_name="s")

    @functools.partial(
        pl.kernel,
        out_type=jax.ShapeDtypeStruct((NC, R, width), jnp.float32),
        mesh=mesh,
        compiler_params=pltpu.CompilerParams(use_tc_tiling_on_sc=False),
        scratch_types=[
            pltpu.VMEM((P,), jnp.int32),
            pltpu.VMEM((CHUNK,), jnp.int32),
            pltpu.VMEM((CHUNK,), jnp.int32),
            pltpu.VMEM((CHUNK, width), jnp.float32),
            pltpu.VMEM((CHUNK, width), jnp.float32),
            pltpu.VMEM_SHARED((R, width), jnp.float32),
            pltpu.SemaphoreType.DMA,
            pltpu.SemaphoreType.DMA,
            pltpu.SemaphoreType.DMA,
            pltpu.SemaphoreType.DMA,
        ],
    )
    def stage(table, gidx, sidx, zeros, out, gi, siA, siB, rows0, rows1,
              acc, semr0, semr1, semi0, semi1):
        cid = lax.axis_index("c")
        sid = lax.axis_index("s")
        wid = cid * NS + sid
        pltpu.sync_copy(zeros.at[pl.ds(sid * rz, rz)], acc.at[pl.ds(sid * rz, rz)])
        base0 = wid * P
        pltpu.sync_copy(gidx.at[pl.ds(base0, P)], gi)
        plsc.subcore_barrier()

        def start_gather(j, rbuf, sem):
            pltpu.async_copy(table.at[gi.at[pl.ds(j * CHUNK, CHUNK)]],
                             rbuf, sem)

        def wait_gather(j, rbuf, sem):
            pltpu.make_async_copy(table.at[gi.at[pl.ds(j * CHUNK, CHUNK)]],
                                  rbuf, sem).wait()

        def start_sidx(j, sbuf, sem):
            pltpu.async_copy(sidx.at[pl.ds(base0 + j * CHUNK, CHUNK)],
                             sbuf, sem)

        def wait_sidx(j, sbuf, sem):
            pltpu.make_async_copy(sidx.at[pl.ds(base0 + j * CHUNK, CHUNK)],
                                  sbuf, sem).wait()

        def scatter(j, rbuf, sbuf):
            pltpu.sync_copy(rbuf, acc.at[sbuf], add=True)

        start_gather(0, rows0, semr0)
        start_sidx(0, siA, semi0)

        def body(k, carry):
            j0 = 2 * k
            wait_gather(j0, rows0, semr0)
            start_gather(j0 + 1, rows1, semr1)
            start_sidx(j0 + 1, siB, semi1)
            wait_sidx(j0, siA, semi0)
            scatter(j0, rows0, siA)
            wait_gather(j0 + 1, rows1, semr1)
            start_gather(j0 + 2, rows0, semr0)
            start_sidx(j0 + 2, siA, semi0)
            wait_sidx(j0 + 1, siB, semi1)
            scatter(j0 + 1, rows1, siB)
            return carry

        lax.fori_loop(0, (nchunk - 1) // 2, body, 0)
        wait_gather(nchunk - 1, rows0, semr0)
        wait_sidx(nchunk - 1, siA, semi0)
        scatter(nchunk - 1, rows0, siA)
        plsc.subcore_barrier()
        pltpu.sync_copy(acc.at[pl.ds(sid * rz, rz)],
                        out.at[cid, pl.ds(sid * rz, rz)])

    return stage


def kernel(X, vertex, edges, degE, degV, W):
    N, D = X.shape
    OUT = W.shape[0]
    M = degE.shape[0]
    NNZ = vertex.shape[0]
    DA = OUT + 16

    xp_aug = _tc_linear_aug(X, W.T, DA)

    zA = jnp.zeros((M, DA), jnp.float32)
    pA = _make_sc_stage(M, DA, NNZ)(xp_aug, vertex, edges, zA)
    xe = _tc_mean_scale(pA, degE, OUT)

    zB = jnp.zeros((N, OUT), jnp.float32)
    pB = _make_sc_stage(N, OUT, NNZ)(xe, edges, vertex, zB)
    return _tc_scale(pB, degV)
